# baseline (speedup 1.0000x reference)
import functools

import jax
import jax.numpy as jnp
from jax import lax
from jax.experimental import pallas as pl
from jax.experimental.pallas import tpu as pltpu
from jax.experimental.pallas import tpu_sc as plsc

DIM = 64
TILE_ROWS = 8
NUM_CORES = 2
NUM_SUBCORES = 16
LANES = 16
NUM_WORKERS = NUM_CORES * NUM_SUBCORES
CHUNK = 64


def _make_kernel(batch):
    b_per_w = batch // NUM_WORKERS
    n_chunks = b_per_w // CHUNK
    mesh = plsc.VectorSubcoreMesh(core_axis_name="c", subcore_axis_name="s")

    @functools.partial(
        pl.kernel,
        mesh=mesh,
        compiler_params=pltpu.CompilerParams(
            needs_layout_passes=False, use_tc_tiling_on_sc=False),
        out_type=jax.ShapeDtypeStruct((batch,), jnp.float32),
        scratch_types=[
            pltpu.VMEM((b_per_w,), jnp.int32),
            pltpu.VMEM((b_per_w,), jnp.int32),
            pltpu.VMEM((b_per_w,), jnp.int32),
            pltpu.VMEM((b_per_w,), jnp.int32),
            pltpu.VMEM((CHUNK, TILE_ROWS, DIM), jnp.float32),
            pltpu.VMEM((CHUNK, TILE_ROWS, DIM), jnp.float32),
            pltpu.VMEM((b_per_w,), jnp.float32),
            pltpu.SemaphoreType.DMA,
            pltpu.SemaphoreType.DMA,
        ],
    )
    def word2vec_sc(center_hbm, context_hbm, inw_hbm, outw_hbm, out_hbm,
                    ctid_v, xtid_v, crem_v, xrem_v, v_slab, u_slab, res_v,
                    sem_v, sem_u):
        wid = lax.axis_index("s") * NUM_CORES + lax.axis_index("c")
        base = wid * b_per_w

        pltpu.sync_copy(center_hbm.at[pl.ds(base, b_per_w)], ctid_v)
        pltpu.sync_copy(context_hbm.at[pl.ds(base, b_per_w)], xtid_v)

        def split_body(g, _):
            s = pl.ds(g * LANES, LANES)
            ci = ctid_v[s]
            xi = xtid_v[s]
            crem_v[s] = ci & (TILE_ROWS - 1)
            xrem_v[s] = xi & (TILE_ROWS - 1)
            ctid_v[s] = ci >> 3
            xtid_v[s] = xi >> 3
            return 0

        lax.fori_loop(0, b_per_w // LANES, split_body, 0)

        lane = lax.broadcasted_iota(jnp.int32, (LANES,), 0)

        def chunk_body(k, _):
            cbase = k * CHUNK
            cp_v = pltpu.async_copy(
                inw_hbm.at[ctid_v.at[pl.ds(cbase, CHUNK)]], v_slab, sem_v)
            cp_u = pltpu.async_copy(
                outw_hbm.at[xtid_v.at[pl.ds(cbase, CHUNK)]], u_slab, sem_u)
            cp_v.wait()
            cp_u.wait()
            for g in range(CHUNK // LANES):
                s = pl.ds(cbase + g * LANES, LANES)
                slab16 = g * LANES + lane
                rv = crem_v[s]
                ru = xrem_v[s]
                acc = jnp.zeros((LANES,), jnp.float32)
                for d in range(DIM):
                    dd = jnp.full((LANES,), d, jnp.int32)
                    vv = plsc.load_gather(v_slab, [slab16, rv, dd])
                    uu = plsc.load_gather(u_slab, [slab16, ru, dd])
                    acc = acc + vv * uu
                res_v[s] = acc
            return 0

        lax.fori_loop(0, n_chunks, chunk_body, 0)
        pltpu.sync_copy(res_v, out_hbm.at[pl.ds(base, b_per_w)])

    return word2vec_sc


def kernel(center_idx, context_idx, in_weight, out_weight):
    batch = center_idx.shape[0]
    vocab = in_weight.shape[0]
    fn = _make_kernel(batch)
    inw3 = in_weight.reshape(vocab // TILE_ROWS, TILE_ROWS, DIM)
    outw3 = out_weight.reshape(vocab // TILE_ROWS, TILE_ROWS, DIM)
    return fn(center_idx.astype(jnp.int32), context_idx.astype(jnp.int32),
              inw3, outw3)


# speedup vs baseline: 2.5254x; 2.5254x over previous
import functools

import jax
import jax.numpy as jnp
from jax import lax
from jax.experimental import pallas as pl
from jax.experimental.pallas import tpu as pltpu
from jax.experimental.pallas import tpu_sc as plsc

DIM = 64
TILE_ROWS = 8
NUM_CORES = 2
NUM_SUBCORES = 16
LANES = 16
NUM_WORKERS = NUM_CORES * NUM_SUBCORES
CHUNK = 256
FIRE_UNROLL = 16


def _make_kernel(batch):
    b_per_w = batch // NUM_WORKERS
    n_chunks = b_per_w // CHUNK
    n_slab = CHUNK // TILE_ROWS
    mesh = plsc.VectorSubcoreMesh(core_axis_name="c", subcore_axis_name="s")

    @functools.partial(
        pl.kernel,
        mesh=mesh,
        compiler_params=pltpu.CompilerParams(needs_layout_passes=False),
        out_type=jax.ShapeDtypeStruct((batch,), jnp.float32),
        scratch_types=[
            pltpu.SMEM((b_per_w,), jnp.int32),
            pltpu.SMEM((b_per_w,), jnp.int32),
            pltpu.VMEM((b_per_w,), jnp.int32),
            pltpu.VMEM((n_slab, TILE_ROWS, DIM), jnp.float32),
            pltpu.VMEM((n_slab, TILE_ROWS, DIM), jnp.float32),
            pltpu.VMEM((b_per_w,), jnp.float32),
            pltpu.SemaphoreType.DMA,
            pltpu.SemaphoreType.DMA,
        ],
    )
    def word2vec_sc(center_hbm, context_hbm, inw_hbm, outw_hbm, out_hbm,
                    cidx_s, xidx_s, idx_v, v_rows, u_rows, res_v,
                    sem_v, sem_u):
        wid = lax.axis_index("s") * NUM_CORES + lax.axis_index("c")
        base = wid * b_per_w

        pltpu.sync_copy(center_hbm.at[pl.ds(base, b_per_w)], idx_v)

        def stage_c(g, _):
            vec = idx_v[pl.ds(g * LANES, LANES)]
            for j in range(LANES):
                cidx_s[g * LANES + j] = vec[j]
            return 0

        lax.fori_loop(0, b_per_w // LANES, stage_c, 0)
        pltpu.sync_copy(context_hbm.at[pl.ds(base, b_per_w)], idx_v)

        def stage_x(g, _):
            vec = idx_v[pl.ds(g * LANES, LANES)]
            for j in range(LANES):
                xidx_s[g * LANES + j] = vec[j]
            return 0

        lax.fori_loop(0, b_per_w // LANES, stage_x, 0)

        n_col = DIM // LANES
        lane = lax.broadcasted_iota(jnp.int32, (LANES,), 0)
        lane_masks = [lane == j for j in range(LANES)]

        def chunk_body(k, _):
            cbase = k * CHUNK

            def fire_body(f, _):
                for jj in range(FIRE_UNROLL):
                    j = f * FIRE_UNROLL + jj
                    ic = cidx_s[cbase + j]
                    ix = xidx_s[cbase + j]
                    pltpu.async_copy(
                        inw_hbm.at[ic >> 3, ic & (TILE_ROWS - 1)],
                        v_rows.at[j // TILE_ROWS, j % TILE_ROWS], sem_v)
                    pltpu.async_copy(
                        outw_hbm.at[ix >> 3, ix & (TILE_ROWS - 1)],
                        u_rows.at[j // TILE_ROWS, j % TILE_ROWS], sem_u)
                return 0

            lax.fori_loop(0, CHUNK // FIRE_UNROLL, fire_body, 0)
            pltpu.make_async_copy(
                inw_hbm.at[pl.ds(0, n_slab)], v_rows, sem_v).wait()
            pltpu.make_async_copy(
                outw_hbm.at[pl.ds(0, n_slab)], u_rows, sem_u).wait()

            def group_body(g, _):
                accv = jnp.zeros((LANES,), jnp.float32)
                for j in range(LANES):
                    b = g * LANES + j
                    acc = None
                    for c in range(n_col):
                        vv = v_rows[b // TILE_ROWS, b % TILE_ROWS,
                                    pl.ds(c * LANES, LANES)]
                        uu = u_rows[b // TILE_ROWS, b % TILE_ROWS,
                                    pl.ds(c * LANES, LANES)]
                        p = vv * uu
                        acc = p if acc is None else acc + p
                    accv = jnp.where(lane_masks[j], jnp.sum(acc), accv)
                res_v[pl.ds(cbase + g * LANES, LANES)] = accv
                return 0

            lax.fori_loop(0, CHUNK // LANES, group_body, 0)
            return 0

        lax.fori_loop(0, n_chunks, chunk_body, 0)
        pltpu.sync_copy(res_v, out_hbm.at[pl.ds(base, b_per_w)])

    return word2vec_sc


def kernel(center_idx, context_idx, in_weight, out_weight):
    batch = center_idx.shape[0]
    vocab = in_weight.shape[0]
    fn = _make_kernel(batch)
    inw3 = in_weight.reshape(vocab // TILE_ROWS, TILE_ROWS, DIM)
    outw3 = out_weight.reshape(vocab // TILE_ROWS, TILE_ROWS, DIM)
    return fn(center_idx.astype(jnp.int32), context_idx.astype(jnp.int32),
              inw3, outw3)
